# baseline (speedup 1.0000x reference)
import jax
import jax.numpy as jnp
from jax import lax
from jax.experimental import pallas as pl
from jax.experimental.pallas import tpu as pltpu
from jax.experimental.pallas import tpu_sc as plsc

D_MODEL = 64
NUM_WORKERS = 32
CHUNK = 128
LANES = 16


def _transpose_block(g_v, o_v):
    iota = lax.iota(jnp.int32, LANES)
    rows = [iota + g * LANES for g in range(D_MODEL // LANES)]

    @plsc.parallel_loop(0, CHUNK, unroll=8, carry=jnp.zeros((LANES,), jnp.int32))
    def _(j, j_vec):
        for g in range(D_MODEL // LANES):
            vals = g_v[j, pl.ds(g * LANES, LANES)]
            plsc.store_scatter(o_v, [rows[g], j_vec], vals)
        return j_vec + 1


def _emb_body(idx_hbm, table_hbm, out_hbm, idx_v, g_a, g_b,
              o_a, o_b, gsem_a, gsem_b, ssem_a, ssem_b):
    wid = lax.axis_index("s") * 2 + lax.axis_index("c")
    n_chunks_total = idx_hbm.shape[0]
    chunks_per_w = n_chunks_total // NUM_WORKERS
    chunk0 = wid * chunks_per_w
    n_batch_blocks = out_hbm.shape[2] // CHUNK

    pltpu.sync_copy(idx_hbm.at[pl.ds(chunk0, chunks_per_w)], idx_v)

    def start_gather(buf, sem, c_local):
        pltpu.async_copy(table_hbm.at[idx_v.at[c_local]], buf, sem)

    def wait_gather(buf, sem):
        pltpu.make_async_copy(table_hbm.at[pl.ds(0, CHUNK)], buf, sem).wait()

    def start_store(o_v, sem, c_local):
        c = chunk0 + c_local
        h = c // n_batch_blocks
        b0 = (c % n_batch_blocks) * CHUNK
        for r in range(D_MODEL // 8):
            pltpu.async_copy(
                o_v.at[pl.ds(r * 8, 8)],
                out_hbm.at[h, pl.ds(r * 8, 8), pl.ds(b0, CHUNK)],
                sem,
            )

    def wait_store(o_v, sem):
        pltpu.make_async_copy(
            o_v, out_hbm.at[0, pl.ds(0, D_MODEL), pl.ds(0, CHUNK)], sem
        ).wait()

    start_gather(g_a, gsem_a, 0)

    def body(i, carry):
        ca = 2 * i
        cb = 2 * i + 1

        start_gather(g_b, gsem_b, cb)
        wait_gather(g_a, gsem_a)

        @pl.when(i > 0)
        def _():
            wait_store(o_a, ssem_a)
        _transpose_block(g_a, o_a)
        start_store(o_a, ssem_a, ca)

        @pl.when(i < chunks_per_w // 2 - 1)
        def _():
            start_gather(g_a, gsem_a, ca + 2)
        wait_gather(g_b, gsem_b)

        @pl.when(i > 0)
        def _():
            wait_store(o_b, ssem_b)
        _transpose_block(g_b, o_b)
        start_store(o_b, ssem_b, cb)

        return carry

    lax.fori_loop(0, chunks_per_w // 2, body, 0)
    wait_store(o_a, ssem_a)
    wait_store(o_b, ssem_b)


def _pack_body(t_ref, o_ref):
    t = t_ref[...].T
    o_ref[...] = jnp.concatenate([t, t], axis=1)


def _pack_table(table_t):
    d, v = table_t.shape
    vblk = 512
    return pl.pallas_call(
        _pack_body,
        grid=(pl.cdiv(v, vblk),),
        in_specs=[pl.BlockSpec((d, vblk), lambda i: (0, i))],
        out_specs=pl.BlockSpec((vblk, 2 * d), lambda i: (i, 0)),
        out_shape=jax.ShapeDtypeStruct((v, 2 * d), jnp.float32),
    )(table_t)


@jax.jit
def kernel(x, table):
    b, h = x.shape
    v, d = table.shape
    n_chunks = (b * h) // CHUNK
    xt = x.T.reshape(n_chunks, CHUNK).astype(jnp.int32)
    table2 = _pack_table(table.T)
    mesh = plsc.VectorSubcoreMesh(core_axis_name="c", subcore_axis_name="s")
    gather = pl.kernel(
        _emb_body,
        out_type=jax.ShapeDtypeStruct((h, d, b), jnp.float32),
        mesh=mesh,
        scratch_types=[
            pltpu.VMEM((n_chunks // NUM_WORKERS, CHUNK), jnp.int32),
            pltpu.VMEM((CHUNK, 128), jnp.float32),
            pltpu.VMEM((CHUNK, 128), jnp.float32),
            pltpu.VMEM((d, CHUNK), jnp.float32),
            pltpu.VMEM((d, CHUNK), jnp.float32),
            pltpu.SemaphoreType.DMA,
            pltpu.SemaphoreType.DMA,
            pltpu.SemaphoreType.DMA,
            pltpu.SemaphoreType.DMA,
        ],
        compiler_params=pltpu.CompilerParams(use_tc_tiling_on_sc=True,
                                             needs_layout_passes=False),
    )
    out = gather(xt, table2)
    return out.transpose(2, 0, 1)


# speedup vs baseline: 1.6138x; 1.6138x over previous
import jax
import jax.numpy as jnp
from jax import lax
from jax.experimental import pallas as pl
from jax.experimental.pallas import tpu as pltpu
from jax.experimental.pallas import tpu_sc as plsc

D_MODEL = 64
NUM_WORKERS = 32
CHUNK = 128
LANES = 16


def _transpose_block(g_v, o_v):
    iota = lax.iota(jnp.int32, LANES)
    rows = [iota + g * LANES for g in range(D_MODEL // LANES)]

    @plsc.parallel_loop(0, CHUNK, unroll=8, carry=jnp.zeros((LANES,), jnp.int32))
    def _(j, j_vec):
        for g in range(D_MODEL // LANES):
            vals = g_v[j, pl.ds(g * LANES, LANES)]
            plsc.store_scatter(o_v, [rows[g], j_vec], vals)
        return j_vec + 1


def _emb_body(idx_hbm, table_hbm, out_hbm, idx_v, g_a, g_b,
              o_a, o_b, gsem_a, gsem_b, ssem_a, ssem_b):
    wid = lax.axis_index("s") * 2 + lax.axis_index("c")
    n_chunks_total = idx_hbm.shape[0]
    chunks_per_w = n_chunks_total // NUM_WORKERS
    chunk0 = wid * chunks_per_w
    n_batch_blocks = out_hbm.shape[2] // CHUNK

    pltpu.sync_copy(idx_hbm.at[pl.ds(chunk0, chunks_per_w)], idx_v)

    def start_gather(buf, sem, c_local):
        pltpu.async_copy(table_hbm.at[idx_v.at[c_local]], buf, sem)

    def wait_gather(buf, sem):
        pltpu.make_async_copy(table_hbm.at[pl.ds(0, CHUNK)], buf, sem).wait()

    def start_store(o_v, sem, c_local):
        c = chunk0 + c_local
        h = c // n_batch_blocks
        b0 = (c % n_batch_blocks) * CHUNK
        for r in range(D_MODEL // 8):
            pltpu.async_copy(
                o_v.at[pl.ds(r * 8, 8)],
                out_hbm.at[h, pl.ds(r * 8, 8), pl.ds(b0, CHUNK)],
                sem,
            )

    def wait_store(o_v, sem):
        pltpu.make_async_copy(
            o_v, out_hbm.at[0, pl.ds(0, D_MODEL), pl.ds(0, CHUNK)], sem
        ).wait()

    start_gather(g_a, gsem_a, 0)

    def body(i, carry):
        ca = 2 * i
        cb = 2 * i + 1

        start_gather(g_b, gsem_b, cb)
        wait_gather(g_a, gsem_a)

        @pl.when(i > 0)
        def _():
            wait_store(o_a, ssem_a)
        _transpose_block(g_a, o_a)
        start_store(o_a, ssem_a, ca)

        @pl.when(i < chunks_per_w // 2 - 1)
        def _():
            start_gather(g_a, gsem_a, ca + 2)
        wait_gather(g_b, gsem_b)

        @pl.when(i > 0)
        def _():
            wait_store(o_b, ssem_b)
        _transpose_block(g_b, o_b)
        start_store(o_b, ssem_b, cb)

        return carry

    lax.fori_loop(0, chunks_per_w // 2, body, 0)
    wait_store(o_a, ssem_a)
    wait_store(o_b, ssem_b)


def _pack_body(t_ref, o_ref):
    t = t_ref[...].T
    o_ref[...] = jnp.concatenate([t, t], axis=1)


def _pack_table(table_t):
    d, v = table_t.shape
    vblk = 2048
    return pl.pallas_call(
        _pack_body,
        grid=(pl.cdiv(v, vblk),),
        in_specs=[pl.BlockSpec((d, vblk), lambda i: (0, i))],
        out_specs=pl.BlockSpec((vblk, 2 * d), lambda i: (i, 0)),
        out_shape=jax.ShapeDtypeStruct((v, 2 * d), jnp.float32),
    )(table_t)


@jax.jit
def kernel(x, table):
    b, h = x.shape
    v, d = table.shape
    n_chunks = (b * h) // CHUNK
    xt = x.T.reshape(n_chunks, CHUNK).astype(jnp.int32)
    table2 = _pack_table(table.T)
    mesh = plsc.VectorSubcoreMesh(core_axis_name="c", subcore_axis_name="s")
    gather = pl.kernel(
        _emb_body,
        out_type=jax.ShapeDtypeStruct((h, d, b), jnp.float32),
        mesh=mesh,
        scratch_types=[
            pltpu.VMEM((n_chunks // NUM_WORKERS, CHUNK), jnp.int32),
            pltpu.VMEM((CHUNK, 128), jnp.float32),
            pltpu.VMEM((CHUNK, 128), jnp.float32),
            pltpu.VMEM((d, CHUNK), jnp.float32),
            pltpu.VMEM((d, CHUNK), jnp.float32),
            pltpu.SemaphoreType.DMA,
            pltpu.SemaphoreType.DMA,
            pltpu.SemaphoreType.DMA,
            pltpu.SemaphoreType.DMA,
        ],
        compiler_params=pltpu.CompilerParams(use_tc_tiling_on_sc=True,
                                             needs_layout_passes=False),
    )
    out = gather(xt, table2)
    return out.transpose(2, 0, 1)


# speedup vs baseline: 1.9024x; 1.1788x over previous
import jax
import jax.numpy as jnp
from jax import lax
from jax.experimental import pallas as pl
from jax.experimental.pallas import tpu as pltpu
from jax.experimental.pallas import tpu_sc as plsc

D_MODEL = 64
NUM_WORKERS = 32
CHUNK = 128
LANES = 16


def _transpose_block(g_v, o_v):
    iota = lax.iota(jnp.int32, LANES)
    rows = [iota + g * LANES for g in range(D_MODEL // LANES)]

    @plsc.parallel_loop(0, CHUNK, unroll=8, carry=jnp.zeros((LANES,), jnp.int32))
    def _(j, j_vec):
        for g in range(D_MODEL // LANES):
            vals = g_v[j, pl.ds(g * LANES, LANES)]
            plsc.store_scatter(o_v, [rows[g], j_vec], vals)
        return j_vec + 1


def _emb_body(idx_hbm, table_hbm, out_hbm, idx_v, g_a, g_b,
              o_a, o_b, gsem_a, gsem_b, ssem_a, ssem_b):
    wid = lax.axis_index("s") * 2 + lax.axis_index("c")
    n_chunks_total = idx_hbm.shape[0]
    chunks_per_w = n_chunks_total // NUM_WORKERS
    chunk0 = wid * chunks_per_w
    n_batch_blocks = out_hbm.shape[2] // CHUNK

    pltpu.sync_copy(idx_hbm.at[pl.ds(chunk0, chunks_per_w)], idx_v)

    def start_gather(buf, sem, c_local):
        pltpu.async_copy(table_hbm.at[idx_v.at[c_local]], buf, sem)

    def wait_gather(buf, sem):
        pltpu.make_async_copy(table_hbm.at[pl.ds(0, CHUNK)], buf, sem).wait()

    def start_store(o_v, sem, c_local):
        c = chunk0 + c_local
        h = c // n_batch_blocks
        b0 = (c % n_batch_blocks) * CHUNK
        for r in range(D_MODEL // 8):
            pltpu.async_copy(
                o_v.at[pl.ds(r * 8, 8)],
                out_hbm.at[h, pl.ds(r * 8, 8), pl.ds(b0, CHUNK)],
                sem,
            )

    def wait_store(o_v, sem):
        pltpu.make_async_copy(
            o_v, out_hbm.at[0, pl.ds(0, D_MODEL), pl.ds(0, CHUNK)], sem
        ).wait()

    start_gather(g_a, gsem_a, 0)

    def body(i, carry):
        ca = 2 * i
        cb = 2 * i + 1

        start_gather(g_b, gsem_b, cb)
        wait_gather(g_a, gsem_a)

        @pl.when(i > 0)
        def _():
            wait_store(o_a, ssem_a)
        _transpose_block(g_a, o_a)
        start_store(o_a, ssem_a, ca)

        @pl.when(i < chunks_per_w // 2 - 1)
        def _():
            start_gather(g_a, gsem_a, ca + 2)
        wait_gather(g_b, gsem_b)

        @pl.when(i > 0)
        def _():
            wait_store(o_b, ssem_b)
        _transpose_block(g_b, o_b)
        start_store(o_b, ssem_b, cb)

        return carry

    lax.fori_loop(0, chunks_per_w // 2, body, 0)
    wait_store(o_a, ssem_a)
    wait_store(o_b, ssem_b)


def _pack_body(t_ref, o_ref):
    t = t_ref[...].T
    o_ref[...] = jnp.concatenate([t, t], axis=1)


def _pack_table(table_t):
    d, v = table_t.shape
    vblk = 8192
    return pl.pallas_call(
        _pack_body,
        grid=(pl.cdiv(v, vblk),),
        in_specs=[pl.BlockSpec((d, vblk), lambda i: (0, i))],
        out_specs=pl.BlockSpec((vblk, 2 * d), lambda i: (i, 0)),
        out_shape=jax.ShapeDtypeStruct((v, 2 * d), jnp.float32),
    )(table_t)


@jax.jit
def kernel(x, table):
    b, h = x.shape
    v, d = table.shape
    n_chunks = (b * h) // CHUNK
    xt = x.T.reshape(n_chunks, CHUNK).astype(jnp.int32)
    table2 = _pack_table(table.T)
    mesh = plsc.VectorSubcoreMesh(core_axis_name="c", subcore_axis_name="s")
    gather = pl.kernel(
        _emb_body,
        out_type=jax.ShapeDtypeStruct((h, d, b), jnp.float32),
        mesh=mesh,
        scratch_types=[
            pltpu.VMEM((n_chunks // NUM_WORKERS, CHUNK), jnp.int32),
            pltpu.VMEM((CHUNK, 128), jnp.float32),
            pltpu.VMEM((CHUNK, 128), jnp.float32),
            pltpu.VMEM((d, CHUNK), jnp.float32),
            pltpu.VMEM((d, CHUNK), jnp.float32),
            pltpu.SemaphoreType.DMA,
            pltpu.SemaphoreType.DMA,
            pltpu.SemaphoreType.DMA,
            pltpu.SemaphoreType.DMA,
        ],
        compiler_params=pltpu.CompilerParams(use_tc_tiling_on_sc=True,
                                             needs_layout_passes=False),
    )
    out = gather(xt, table2)
    return out.transpose(2, 0, 1)


# speedup vs baseline: 1.9650x; 1.0329x over previous
import jax
import jax.numpy as jnp
from jax import lax
from jax.experimental import pallas as pl
from jax.experimental.pallas import tpu as pltpu
from jax.experimental.pallas import tpu_sc as plsc

D_MODEL = 64
NUM_WORKERS = 32
CHUNK = 128
LANES = 16


def _transpose_block(g_v, o_v):
    iota = lax.iota(jnp.int32, LANES)
    rows = [iota + g * LANES for g in range(D_MODEL // LANES)]

    @plsc.parallel_loop(0, CHUNK, unroll=8, carry=jnp.zeros((LANES,), jnp.int32))
    def _(j, j_vec):
        for g in range(D_MODEL // LANES):
            vals = g_v[j, pl.ds(g * LANES, LANES)]
            plsc.store_scatter(o_v, [rows[g], j_vec], vals)
        return j_vec + 1


def _emb_body(idx_hbm, table_hbm, out_hbm, idx_v, g_a, g_b,
              o_a, o_b, gsem_a, gsem_b, ssem_a, ssem_b):
    wid = lax.axis_index("s") * 2 + lax.axis_index("c")
    n_chunks_total = idx_hbm.shape[0]
    chunks_per_w = n_chunks_total // NUM_WORKERS
    chunk0 = wid * chunks_per_w
    n_batch_blocks = out_hbm.shape[2] // CHUNK

    pltpu.sync_copy(idx_hbm.at[pl.ds(chunk0, chunks_per_w)], idx_v)

    def start_gather(buf, sem, c_local):
        pltpu.async_copy(table_hbm.at[idx_v.at[c_local]], buf, sem)

    def wait_gather(buf, sem):
        pltpu.make_async_copy(table_hbm.at[pl.ds(0, CHUNK)], buf, sem).wait()

    def start_store(o_v, sem, c_local):
        c = chunk0 + c_local
        h = c // n_batch_blocks
        b0 = (c % n_batch_blocks) * CHUNK
        for r in range(D_MODEL // 8):
            pltpu.async_copy(
                o_v.at[pl.ds(r * 8, 8)],
                out_hbm.at[h, pl.ds(r * 8, 8), pl.ds(b0, CHUNK)],
                sem,
            )

    def wait_store(o_v, sem):
        pltpu.make_async_copy(
            o_v, out_hbm.at[0, pl.ds(0, D_MODEL), pl.ds(0, CHUNK)], sem
        ).wait()

    start_gather(g_a, gsem_a, 0)

    def body(i, carry):
        ca = 2 * i
        cb = 2 * i + 1

        start_gather(g_b, gsem_b, cb)
        wait_gather(g_a, gsem_a)

        @pl.when(i > 0)
        def _():
            wait_store(o_a, ssem_a)
        _transpose_block(g_a, o_a)
        start_store(o_a, ssem_a, ca)

        @pl.when(i < chunks_per_w // 2 - 1)
        def _():
            start_gather(g_a, gsem_a, ca + 2)
        wait_gather(g_b, gsem_b)

        @pl.when(i > 0)
        def _():
            wait_store(o_b, ssem_b)
        _transpose_block(g_b, o_b)
        start_store(o_b, ssem_b, cb)

        return carry

    lax.fori_loop(0, chunks_per_w // 2, body, 0)
    wait_store(o_a, ssem_a)
    wait_store(o_b, ssem_b)


def _pack_body(t_ref, o_ref):
    t = t_ref[...].T
    o_ref[...] = jnp.concatenate([t, t], axis=1)


def _pack_table(table_t):
    d, v = table_t.shape
    vblk = 16384
    return pl.pallas_call(
        _pack_body,
        grid=(pl.cdiv(v, vblk),),
        in_specs=[pl.BlockSpec((d, vblk), lambda i: (0, i))],
        out_specs=pl.BlockSpec((vblk, 2 * d), lambda i: (i, 0)),
        out_shape=jax.ShapeDtypeStruct((v, 2 * d), jnp.float32),
    )(table_t)


@jax.jit
def kernel(x, table):
    b, h = x.shape
    v, d = table.shape
    n_chunks = (b * h) // CHUNK
    xt = x.T.reshape(n_chunks, CHUNK).astype(jnp.int32)
    table2 = _pack_table(table.T)
    mesh = plsc.VectorSubcoreMesh(core_axis_name="c", subcore_axis_name="s")
    gather = pl.kernel(
        _emb_body,
        out_type=jax.ShapeDtypeStruct((h, d, b), jnp.float32),
        mesh=mesh,
        scratch_types=[
            pltpu.VMEM((n_chunks // NUM_WORKERS, CHUNK), jnp.int32),
            pltpu.VMEM((CHUNK, 128), jnp.float32),
            pltpu.VMEM((CHUNK, 128), jnp.float32),
            pltpu.VMEM((d, CHUNK), jnp.float32),
            pltpu.VMEM((d, CHUNK), jnp.float32),
            pltpu.SemaphoreType.DMA,
            pltpu.SemaphoreType.DMA,
            pltpu.SemaphoreType.DMA,
            pltpu.SemaphoreType.DMA,
        ],
        compiler_params=pltpu.CompilerParams(use_tc_tiling_on_sc=True,
                                             needs_layout_passes=False),
    )
    out = gather(xt, table2)
    return out.transpose(2, 0, 1)
